# Initial kernel scaffold; baseline (speedup 1.0000x reference)
import jax
import jax.numpy as jnp
from jax.experimental import pallas as pl


def kernel(x, edge_index, edge_attr, nn1_w1, nn1_b1, nn1_w2, nn1_b2, root1, bias1, nn2_w1, nn2_b1, nn2_w2, nn2_b2, root2, bias2):
    raise NotImplementedError("write your pallas kernel here")


# speedup vs baseline: 1.0853x; 1.0853x over previous
import functools
import jax
import jax.numpy as jnp
from jax import lax
from jax.experimental import pallas as pl
from jax.experimental.pallas import tpu as pltpu

N = 10000
E = 160000
IN = 16
HID = 16
OUT = 16
ED = 16
H = 128

BE = 2000


def _msg_body(ea_ref, xs_ref, w1_ref, b1_ref, w2_ref, b2_ref, p_ref, q_ref,
              out_ref):
    ea = ea_ref[...]
    xs = xs_ref[...]
    h = jnp.maximum(
        jnp.dot(ea, w1_ref[...], preferred_element_type=jnp.float32)
        + b1_ref[...], 0.0)
    wf = jnp.dot(h, w2_ref[...], preferred_element_type=jnp.float32) \
        + b2_ref[...]
    xr = jnp.dot(xs, p_ref[...], preferred_element_type=jnp.float32)
    out_ref[...] = jnp.dot(xr * wf, q_ref[...],
                           preferred_element_type=jnp.float32)


def _edge_messages(ea, xs, w1, b1, w2, b2, ic, oc, interpret=False):
    ep = ea.shape[0]
    grid = ep // BE
    ii = jnp.arange(ic * oc) // oc
    oo = jnp.arange(ic * oc) % oc
    p_mat = (ii[None, :] == jnp.arange(ic)[:, None]).astype(jnp.float32)
    q_mat = (oo[:, None] == jnp.arange(oc)[None, :]).astype(jnp.float32)
    b1r = b1.reshape(1, -1)
    b2r = b2.reshape(1, -1)

    def fixed(a):
        return pl.BlockSpec(a.shape, lambda i: (0,) * a.ndim)

    return pl.pallas_call(
        _msg_body,
        grid=(grid,),
        in_specs=[
            pl.BlockSpec((BE, ic), lambda i: (i, 0)),
            pl.BlockSpec((BE, ic), lambda i: (i, 0)),
            fixed(w1), fixed(b1r), fixed(w2), fixed(b2r),
            fixed(p_mat), fixed(q_mat),
        ],
        out_specs=pl.BlockSpec((BE, oc), lambda i: (i, 0)),
        out_shape=jax.ShapeDtypeStruct((ep, oc), jnp.float32),
        interpret=interpret,
    )(ea, xs, w1, b1r, w2, b2r, p_mat, q_mat)


def _node_body(x_ref, sum_ref, cnt_ref, root_ref, bias_ref, out_ref, *,
               do_relu):
    s = sum_ref[0] + sum_ref[1]
    c = cnt_ref[0] + cnt_ref[1]
    inv = 1.0 / jnp.maximum(c, 1.0)
    r = jnp.dot(x_ref[...], root_ref[...], preferred_element_type=jnp.float32)
    o = r + s * inv + bias_ref[...]
    if do_relu:
        o = jnp.maximum(o, 0.0)
    out_ref[...] = o


def _node_combine(x, summed, cnt, root, bias, do_relu, interpret=False):
    n, f = x.shape[0], root.shape[1]
    return pl.pallas_call(
        functools.partial(_node_body, do_relu=do_relu),
        out_shape=jax.ShapeDtypeStruct((n, f), jnp.float32),
        interpret=interpret,
    )(x, summed[:, :n], cnt[:, :n], root, bias)


def kernel(x, edge_index, edge_attr, nn1_w1, nn1_b1, nn1_w2, nn1_b2, root1,
           bias1, nn2_w1, nn2_b1, nn2_w2, nn2_b2, root2, bias2):
    src = edge_index[0].astype(jnp.int32)
    dst = edge_index[1].astype(jnp.int32)

    xs = x[src]
    msg1 = _edge_messages(edge_attr, xs, nn1_w1, nn1_b1, nn1_w2, nn1_b2,
                          IN, HID)
    s1 = jax.ops.segment_sum(msg1, dst, num_segments=N)
    c1 = jax.ops.segment_sum(jnp.ones((E, HID), jnp.float32), dst,
                             num_segments=N)
    h = _node_combine(x, jnp.stack([s1, jnp.zeros_like(s1)]),
                      jnp.stack([c1, jnp.zeros_like(c1)]), root1, bias1, True)

    hs = h[src]
    msg2 = _edge_messages(edge_attr, hs, nn2_w1, nn2_b1, nn2_w2, nn2_b2,
                          HID, OUT)
    s2 = jax.ops.segment_sum(msg2, dst, num_segments=N)
    out = _node_combine(h, jnp.stack([s2, jnp.zeros_like(s2)]),
                        jnp.stack([c1, jnp.zeros_like(c1)]), root2, bias2,
                        False)
    return out


# speedup vs baseline: 3.0947x; 2.8516x over previous
import functools
import jax
import jax.numpy as jnp
from jax import lax
from jax.experimental import pallas as pl
from jax.experimental.pallas import tpu as pltpu
from jax.experimental.pallas import tpu_sc as plsc

N = 10000
E = 160000
IN = 16
HID = 16
OUT = 16
ED = 16
H = 128

NC = 2
NS = 16
NW = NC * NS
CH = 128
EP = 163840
EPW = EP // NW
NCH = EPW // CH
RPS = 632
NP = NS * RPS

BE = 2048

_MESH = plsc.VectorSubcoreMesh(core_axis_name="c", subcore_axis_name="s")
_SC_PARAMS = pltpu.CompilerParams(use_tc_tiling_on_sc=False)


@functools.partial(
    pl.kernel,
    out_type=jax.ShapeDtypeStruct((EP, 16), jnp.float32),
    mesh=_MESH,
    scratch_types=[
        pltpu.VMEM((NCH, CH), jnp.int32),
        pltpu.VMEM((EPW, 16), jnp.float32),
        pltpu.SemaphoreType.DMA,
    ],
    compiler_params=_SC_PARAMS,
)
def _sc_gather(tab_hbm, idx_hbm, out_hbm, idx_v, rows_v, sem):
    c = lax.axis_index("c")
    s = lax.axis_index("s")
    wid = s * NC + c
    pltpu.sync_copy(idx_hbm.at[pl.ds(wid * NCH, NCH)], idx_v)

    def jbody(j, carry):
        pltpu.async_copy(tab_hbm.at[idx_v.at[j]],
                         rows_v.at[pl.ds(j * CH, CH)], sem).wait()
        return carry

    lax.fori_loop(0, NCH, jbody, 0)
    pltpu.sync_copy(rows_v, out_hbm.at[pl.ds(wid * EPW, EPW)])


def _scatter_body(idx_hbm, vals_hbm, zeros_hbm, ones_hbm, out_hbm, cout_hbm,
                  idx_v, vals_v, ones_v, acc_sh, cacc_sh, *, with_cnt):
    c = lax.axis_index("c")
    s = lax.axis_index("s")
    wid = s * NC + c
    pltpu.sync_copy(zeros_hbm.at[pl.ds(s * RPS, RPS)],
                    acc_sh.at[pl.ds(s * RPS, RPS)])
    if with_cnt:
        pltpu.sync_copy(zeros_hbm.at[pl.ds(s * RPS, RPS)],
                        cacc_sh.at[pl.ds(s * RPS, RPS)])
        pltpu.sync_copy(ones_hbm, ones_v)
    pltpu.sync_copy(idx_hbm.at[pl.ds(wid * NCH, NCH)], idx_v)
    pltpu.sync_copy(vals_hbm.at[pl.ds(wid * EPW, EPW)], vals_v)
    plsc.subcore_barrier()

    def jbody(j, carry):
        pltpu.sync_copy(vals_v.at[pl.ds(j * CH, CH)],
                        acc_sh.at[idx_v.at[j]], add=True)
        if with_cnt:
            pltpu.sync_copy(ones_v, cacc_sh.at[idx_v.at[j]], add=True)
        return carry

    lax.fori_loop(0, NCH, jbody, 0)
    plsc.subcore_barrier()
    pltpu.sync_copy(acc_sh.at[pl.ds(s * RPS, RPS)],
                    out_hbm.at[pl.ds(c * NP + s * RPS, RPS)])
    if with_cnt:
        pltpu.sync_copy(cacc_sh.at[pl.ds(s * RPS, RPS)],
                        cout_hbm.at[pl.ds(c * NP + s * RPS, RPS)])


@functools.partial(
    pl.kernel,
    out_type=(jax.ShapeDtypeStruct((NC * NP, 16), jnp.float32),
              jax.ShapeDtypeStruct((NC * NP, 16), jnp.float32)),
    mesh=_MESH,
    scratch_types=[
        pltpu.VMEM((NCH, CH), jnp.int32),
        pltpu.VMEM((EPW, 16), jnp.float32),
        pltpu.VMEM((CH, 16), jnp.float32),
        pltpu.VMEM_SHARED((NP, 16), jnp.float32),
        pltpu.VMEM_SHARED((NP, 16), jnp.float32),
    ],
    compiler_params=_SC_PARAMS,
)
def _sc_scatter_cnt(idx_hbm, vals_hbm, zeros_hbm, ones_hbm, out_hbm,
                    cout_hbm, idx_v, vals_v, ones_v, acc_sh, cacc_sh):
    _scatter_body(idx_hbm, vals_hbm, zeros_hbm, ones_hbm, out_hbm, cout_hbm,
                  idx_v, vals_v, ones_v, acc_sh, cacc_sh, with_cnt=True)


@functools.partial(
    pl.kernel,
    out_type=jax.ShapeDtypeStruct((NC * NP, 16), jnp.float32),
    mesh=_MESH,
    scratch_types=[
        pltpu.VMEM((NCH, CH), jnp.int32),
        pltpu.VMEM((EPW, 16), jnp.float32),
        pltpu.VMEM_SHARED((NP, 16), jnp.float32),
    ],
    compiler_params=_SC_PARAMS,
)
def _sc_scatter(idx_hbm, vals_hbm, zeros_hbm, out_hbm, idx_v, vals_v, acc_sh):
    _scatter_body(idx_hbm, vals_hbm, zeros_hbm, None, out_hbm, None,
                  idx_v, vals_v, None, acc_sh, None, with_cnt=False)


def _msg_body(ea_ref, xs_ref, w1_ref, b1_ref, w2_ref, b2_ref, p_ref, q_ref,
              out_ref):
    ea = ea_ref[...]
    xs = xs_ref[...]
    h = jnp.maximum(
        jnp.dot(ea, w1_ref[...], preferred_element_type=jnp.float32)
        + b1_ref[...], 0.0)
    wf = jnp.dot(h, w2_ref[...], preferred_element_type=jnp.float32) \
        + b2_ref[...]
    xr = jnp.dot(xs, p_ref[...], preferred_element_type=jnp.float32)
    out_ref[...] = jnp.dot(xr * wf, q_ref[...],
                           preferred_element_type=jnp.float32)


def _edge_messages(ea, xs, w1, b1, w2, b2, ic, oc):
    ep = ea.shape[0]
    grid = ep // BE
    ii = jnp.arange(ic * oc) // oc
    oo = jnp.arange(ic * oc) % oc
    p_mat = (ii[None, :] == jnp.arange(ic)[:, None]).astype(jnp.float32)
    q_mat = (oo[:, None] == jnp.arange(oc)[None, :]).astype(jnp.float32)
    b1r = b1.reshape(1, -1)
    b2r = b2.reshape(1, -1)

    def fixed(a):
        return pl.BlockSpec(a.shape, lambda i: (0,) * a.ndim)

    return pl.pallas_call(
        _msg_body,
        grid=(grid,),
        in_specs=[
            pl.BlockSpec((BE, ic), lambda i: (i, 0)),
            pl.BlockSpec((BE, ic), lambda i: (i, 0)),
            fixed(w1), fixed(b1r), fixed(w2), fixed(b2r),
            fixed(p_mat), fixed(q_mat),
        ],
        out_specs=pl.BlockSpec((BE, oc), lambda i: (i, 0)),
        out_shape=jax.ShapeDtypeStruct((ep, oc), jnp.float32),
    )(ea, xs, w1, b1r, w2, b2r, p_mat, q_mat)


def _node_body(x_ref, sum_ref, cnt_ref, root_ref, bias_ref, out_ref, *,
               do_relu):
    s = sum_ref[0] + sum_ref[1]
    c = cnt_ref[0] + cnt_ref[1]
    inv = 1.0 / jnp.maximum(c, 1.0)
    r = jnp.dot(x_ref[...], root_ref[...], preferred_element_type=jnp.float32)
    o = r + s * inv + bias_ref[...]
    if do_relu:
        o = jnp.maximum(o, 0.0)
    out_ref[...] = o


def _node_combine(x, summed, cnt, root, bias, do_relu):
    n, f = x.shape[0], root.shape[1]
    return pl.pallas_call(
        functools.partial(_node_body, do_relu=do_relu),
        out_shape=jax.ShapeDtypeStruct((n, f), jnp.float32),
    )(x, summed, cnt, root, bias.reshape(1, -1))


def kernel(x, edge_index, edge_attr, nn1_w1, nn1_b1, nn1_w2, nn1_b2, root1,
           bias1, nn2_w1, nn2_b1, nn2_w2, nn2_b2, root2, bias2):
    src = edge_index[0].astype(jnp.int32)
    dst = edge_index[1].astype(jnp.int32)
    pad = EP - E
    src_c = jnp.concatenate([src, jnp.zeros((pad,), jnp.int32)])
    dst_c = jnp.concatenate([dst, jnp.full((pad,), N, jnp.int32)])
    src2 = src_c.reshape(NW * NCH, CH)
    dst2 = dst_c.reshape(NW * NCH, CH)
    ea_pad = jnp.concatenate([edge_attr, jnp.zeros((pad, ED), jnp.float32)])
    zeros_np = jnp.zeros((NP, 16), jnp.float32)
    ones_ch = jnp.ones((CH, 16), jnp.float32)

    xs = _sc_gather(x, src2)
    msg1 = _edge_messages(ea_pad, xs, nn1_w1, nn1_b1, nn1_w2, nn1_b2,
                          IN, HID)
    s1, c1 = _sc_scatter_cnt(dst2, msg1, zeros_np, ones_ch)
    s1 = s1.reshape(NC, NP, 16)[:, :N]
    c1 = c1.reshape(NC, NP, 16)[:, :N]
    h = _node_combine(x, s1, c1, root1, bias1, True)

    hs = _sc_gather(h, src2)
    msg2 = _edge_messages(ea_pad, hs, nn2_w1, nn2_b1, nn2_w2, nn2_b2,
                          HID, OUT)
    s2 = _sc_scatter(dst2, msg2, zeros_np)
    s2 = s2.reshape(NC, NP, 16)[:, :N]
    out = _node_combine(h, s2, c1, root2, bias2, False)
    return out


# speedup vs baseline: 3.2186x; 1.0400x over previous
import functools
import jax
import jax.numpy as jnp
from jax import lax
from jax.experimental import pallas as pl
from jax.experimental.pallas import tpu as pltpu
from jax.experimental.pallas import tpu_sc as plsc

N = 10000
E = 160000
IN = 16
HID = 16
OUT = 16
ED = 16
H = 128

NC = 2
NS = 16
NW = NC * NS
CH = 128
EP = 163840
EPW = EP // NW
NCH = EPW // CH
RPS = 632
NP = NS * RPS

BE = 2048

_MESH = plsc.VectorSubcoreMesh(core_axis_name="c", subcore_axis_name="s")
_SC_PARAMS = pltpu.CompilerParams(use_tc_tiling_on_sc=False)


@functools.partial(
    pl.kernel,
    out_type=jax.ShapeDtypeStruct((EP, 16), jnp.float32),
    mesh=_MESH,
    scratch_types=[
        pltpu.VMEM((NCH, CH), jnp.int32),
        pltpu.VMEM((EPW, 16), jnp.float32),
        pltpu.SemaphoreType.DMA,
    ],
    compiler_params=_SC_PARAMS,
)
def _sc_gather(tab_hbm, idx_hbm, out_hbm, idx_v, rows_v, sem):
    c = lax.axis_index("c")
    s = lax.axis_index("s")
    wid = s * NC + c
    pltpu.sync_copy(idx_hbm.at[pl.ds(wid * NCH, NCH)], idx_v)

    def fire(j, carry):
        pltpu.async_copy(tab_hbm.at[idx_v.at[j]],
                         rows_v.at[pl.ds(j * CH, CH)], sem)
        return carry

    lax.fori_loop(0, NCH, fire, 0)

    def drain(j, carry):
        pltpu.make_async_copy(tab_hbm.at[idx_v.at[0]],
                              rows_v.at[pl.ds(0, CH)], sem).wait()
        return carry

    lax.fori_loop(0, NCH, drain, 0)
    pltpu.sync_copy(rows_v, out_hbm.at[pl.ds(wid * EPW, EPW)])


def _scatter_body(idx_hbm, vals_hbm, zeros_hbm, ones_hbm, out_hbm, cout_hbm,
                  idx_v, vals_v, ones_v, acc_sh, cacc_sh, sem, *, with_cnt):
    c = lax.axis_index("c")
    s = lax.axis_index("s")
    wid = s * NC + c
    pltpu.sync_copy(zeros_hbm.at[pl.ds(s * RPS, RPS)],
                    acc_sh.at[pl.ds(s * RPS, RPS)])
    if with_cnt:
        pltpu.sync_copy(zeros_hbm.at[pl.ds(s * RPS, RPS)],
                        cacc_sh.at[pl.ds(s * RPS, RPS)])
        pltpu.sync_copy(ones_hbm, ones_v)
    pltpu.sync_copy(idx_hbm.at[pl.ds(wid * NCH, NCH)], idx_v)
    pltpu.sync_copy(vals_hbm.at[pl.ds(wid * EPW, EPW)], vals_v)
    plsc.subcore_barrier()

    def fire(j, carry):
        pltpu.async_copy(vals_v.at[pl.ds(j * CH, CH)],
                         acc_sh.at[idx_v.at[j]], sem, add=True)
        if with_cnt:
            pltpu.async_copy(ones_v, cacc_sh.at[idx_v.at[j]], sem, add=True)
        return carry

    lax.fori_loop(0, NCH, fire, 0)
    n_waits = NCH * 2 if with_cnt else NCH

    def drain(j, carry):
        pltpu.make_async_copy(vals_v.at[pl.ds(0, CH)],
                              acc_sh.at[idx_v.at[0]], sem).wait()
        return carry

    lax.fori_loop(0, n_waits, drain, 0)
    plsc.subcore_barrier()
    pltpu.sync_copy(acc_sh.at[pl.ds(s * RPS, RPS)],
                    out_hbm.at[pl.ds(c * NP + s * RPS, RPS)])
    if with_cnt:
        pltpu.sync_copy(cacc_sh.at[pl.ds(s * RPS, RPS)],
                        cout_hbm.at[pl.ds(c * NP + s * RPS, RPS)])


@functools.partial(
    pl.kernel,
    out_type=(jax.ShapeDtypeStruct((NC * NP, 16), jnp.float32),
              jax.ShapeDtypeStruct((NC * NP, 16), jnp.float32)),
    mesh=_MESH,
    scratch_types=[
        pltpu.VMEM((NCH, CH), jnp.int32),
        pltpu.VMEM((EPW, 16), jnp.float32),
        pltpu.VMEM((CH, 16), jnp.float32),
        pltpu.VMEM_SHARED((NP, 16), jnp.float32),
        pltpu.VMEM_SHARED((NP, 16), jnp.float32),
        pltpu.SemaphoreType.DMA,
    ],
    compiler_params=_SC_PARAMS,
)
def _sc_scatter_cnt(idx_hbm, vals_hbm, zeros_hbm, ones_hbm, out_hbm,
                    cout_hbm, idx_v, vals_v, ones_v, acc_sh, cacc_sh, sem):
    _scatter_body(idx_hbm, vals_hbm, zeros_hbm, ones_hbm, out_hbm, cout_hbm,
                  idx_v, vals_v, ones_v, acc_sh, cacc_sh, sem, with_cnt=True)


@functools.partial(
    pl.kernel,
    out_type=jax.ShapeDtypeStruct((NC * NP, 16), jnp.float32),
    mesh=_MESH,
    scratch_types=[
        pltpu.VMEM((NCH, CH), jnp.int32),
        pltpu.VMEM((EPW, 16), jnp.float32),
        pltpu.VMEM_SHARED((NP, 16), jnp.float32),
        pltpu.SemaphoreType.DMA,
    ],
    compiler_params=_SC_PARAMS,
)
def _sc_scatter(idx_hbm, vals_hbm, zeros_hbm, out_hbm, idx_v, vals_v, acc_sh,
                sem):
    _scatter_body(idx_hbm, vals_hbm, zeros_hbm, None, out_hbm, None,
                  idx_v, vals_v, None, acc_sh, None, sem, with_cnt=False)


def _msg_body(ea_ref, xs_ref, w1_ref, b1_ref, w2_ref, b2_ref, p_ref, q_ref,
              out_ref):
    ea = ea_ref[...]
    xs = xs_ref[...]
    h = jnp.maximum(
        jnp.dot(ea, w1_ref[...], preferred_element_type=jnp.float32)
        + b1_ref[...], 0.0)
    wf = jnp.dot(h, w2_ref[...], preferred_element_type=jnp.float32) \
        + b2_ref[...]
    xr = jnp.dot(xs, p_ref[...], preferred_element_type=jnp.float32)
    out_ref[...] = jnp.dot(xr * wf, q_ref[...],
                           preferred_element_type=jnp.float32)


def _edge_messages(ea, xs, w1, b1, w2, b2, ic, oc):
    ep = ea.shape[0]
    grid = ep // BE
    ii = jnp.arange(ic * oc) // oc
    oo = jnp.arange(ic * oc) % oc
    p_mat = (ii[None, :] == jnp.arange(ic)[:, None]).astype(jnp.float32)
    q_mat = (oo[:, None] == jnp.arange(oc)[None, :]).astype(jnp.float32)
    b1r = b1.reshape(1, -1)
    b2r = b2.reshape(1, -1)

    def fixed(a):
        return pl.BlockSpec(a.shape, lambda i: (0,) * a.ndim)

    return pl.pallas_call(
        _msg_body,
        grid=(grid,),
        in_specs=[
            pl.BlockSpec((BE, ic), lambda i: (i, 0)),
            pl.BlockSpec((BE, ic), lambda i: (i, 0)),
            fixed(w1), fixed(b1r), fixed(w2), fixed(b2r),
            fixed(p_mat), fixed(q_mat),
        ],
        out_specs=pl.BlockSpec((BE, oc), lambda i: (i, 0)),
        out_shape=jax.ShapeDtypeStruct((ep, oc), jnp.float32),
    )(ea, xs, w1, b1r, w2, b2r, p_mat, q_mat)


def _node_body(x_ref, sum_ref, cnt_ref, root_ref, bias_ref, out_ref, *,
               do_relu):
    s = sum_ref[0] + sum_ref[1]
    c = cnt_ref[0] + cnt_ref[1]
    inv = 1.0 / jnp.maximum(c, 1.0)
    r = jnp.dot(x_ref[...], root_ref[...], preferred_element_type=jnp.float32)
    o = r + s * inv + bias_ref[...]
    if do_relu:
        o = jnp.maximum(o, 0.0)
    out_ref[...] = o


def _node_combine(x, summed, cnt, root, bias, do_relu):
    n, f = x.shape[0], root.shape[1]
    return pl.pallas_call(
        functools.partial(_node_body, do_relu=do_relu),
        out_shape=jax.ShapeDtypeStruct((n, f), jnp.float32),
    )(x, summed, cnt, root, bias.reshape(1, -1))


def kernel(x, edge_index, edge_attr, nn1_w1, nn1_b1, nn1_w2, nn1_b2, root1,
           bias1, nn2_w1, nn2_b1, nn2_w2, nn2_b2, root2, bias2):
    src = edge_index[0].astype(jnp.int32)
    dst = edge_index[1].astype(jnp.int32)
    pad = EP - E
    src_c = jnp.concatenate([src, jnp.zeros((pad,), jnp.int32)])
    dst_c = jnp.concatenate([dst, jnp.full((pad,), N, jnp.int32)])
    src2 = src_c.reshape(NW * NCH, CH)
    dst2 = dst_c.reshape(NW * NCH, CH)
    ea_pad = jnp.concatenate([edge_attr, jnp.zeros((pad, ED), jnp.float32)])
    zeros_np = jnp.zeros((NP, 16), jnp.float32)
    ones_ch = jnp.ones((CH, 16), jnp.float32)

    xs = _sc_gather(x, src2)
    msg1 = _edge_messages(ea_pad, xs, nn1_w1, nn1_b1, nn1_w2, nn1_b2,
                          IN, HID)
    s1, c1 = _sc_scatter_cnt(dst2, msg1, zeros_np, ones_ch)
    s1 = s1.reshape(NC, NP, 16)[:, :N]
    c1 = c1.reshape(NC, NP, 16)[:, :N]
    h = _node_combine(x, s1, c1, root1, bias1, True)

    hs = _sc_gather(h, src2)
    msg2 = _edge_messages(ea_pad, hs, nn2_w1, nn2_b1, nn2_w2, nn2_b2,
                          HID, OUT)
    s2 = _sc_scatter(dst2, msg2, zeros_np)
    s2 = s2.reshape(NC, NP, 16)[:, :N]
    out = _node_combine(h, s2, c1, root2, bias2, False)
    return out


# speedup vs baseline: 3.6950x; 1.1480x over previous
import functools
import jax
import jax.numpy as jnp
from jax import lax
from jax.experimental import pallas as pl
from jax.experimental.pallas import tpu as pltpu
from jax.experimental.pallas import tpu_sc as plsc

N = 10000
E = 160000
IN = 16
HID = 16
OUT = 16
ED = 16
H = 128

NC = 2
NS = 16
NW = NC * NS
CH = 128
NCHT = E // CH
NCHW = NCHT // NW
NXTRA = NCHT - NCHW * NW
RPS = 632
NP = NS * RPS

BE = 2000

_MESH = plsc.VectorSubcoreMesh(core_axis_name="c", subcore_axis_name="s")
_SC_PARAMS = pltpu.CompilerParams(use_tc_tiling_on_sc=False)


@functools.partial(
    pl.kernel,
    out_type=jax.ShapeDtypeStruct((E, 16), jnp.float32),
    mesh=_MESH,
    scratch_types=[
        pltpu.VMEM((NCHW + 1, CH), jnp.int32),
        pltpu.VMEM(((NCHW + 1) * CH, 16), jnp.float32),
        pltpu.SemaphoreType.DMA,
    ],
    compiler_params=_SC_PARAMS,
)
def _sc_gather(tab_hbm, idx_hbm, out_hbm, idx_v, rows_v, sem):
    c = lax.axis_index("c")
    s = lax.axis_index("s")
    wid = s * NC + c
    extra = jnp.where(wid < NXTRA, 1, 0)
    cbase = wid * NCHW + jnp.minimum(wid, NXTRA)
    rbase = cbase * CH
    nch = NCHW + extra
    pltpu.sync_copy(idx_hbm.at[pl.ds(cbase, NCHW)], idx_v.at[pl.ds(0, NCHW)])

    @pl.when(extra == 1)
    def _():
        pltpu.sync_copy(idx_hbm.at[pl.ds(cbase + NCHW, 1)],
                        idx_v.at[pl.ds(NCHW, 1)])

    def fire(j, carry):
        pltpu.async_copy(tab_hbm.at[idx_v.at[j]],
                         rows_v.at[pl.ds(j * CH, CH)], sem)
        return carry

    lax.fori_loop(0, nch, fire, 0)

    def drain(j, carry):
        pltpu.make_async_copy(tab_hbm.at[idx_v.at[0]],
                              rows_v.at[pl.ds(0, CH)], sem).wait()
        return carry

    lax.fori_loop(0, nch, drain, 0)
    pltpu.sync_copy(rows_v.at[pl.ds(0, NCHW * CH)],
                    out_hbm.at[pl.ds(rbase, NCHW * CH)])

    @pl.when(extra == 1)
    def _():
        pltpu.sync_copy(rows_v.at[pl.ds(NCHW * CH, CH)],
                        out_hbm.at[pl.ds(rbase + NCHW * CH, CH)])


def _scatter_body(idx_hbm, vals_hbm, zeros_hbm, ones_hbm, out_hbm, cout_hbm,
                  idx_v, vals_v, ones_v, acc_sh, cacc_sh, sem, *, with_cnt):
    c = lax.axis_index("c")
    s = lax.axis_index("s")
    wid = s * NC + c
    extra = jnp.where(wid < NXTRA, 1, 0)
    cbase = wid * NCHW + jnp.minimum(wid, NXTRA)
    rbase = cbase * CH
    nch = NCHW + extra
    pltpu.sync_copy(zeros_hbm.at[pl.ds(s * RPS, RPS)],
                    acc_sh.at[pl.ds(s * RPS, RPS)])
    if with_cnt:
        pltpu.sync_copy(zeros_hbm.at[pl.ds(s * RPS, RPS)],
                        cacc_sh.at[pl.ds(s * RPS, RPS)])
        pltpu.sync_copy(ones_hbm, ones_v)
    pltpu.sync_copy(idx_hbm.at[pl.ds(cbase, NCHW)], idx_v.at[pl.ds(0, NCHW)])
    pltpu.sync_copy(vals_hbm.at[pl.ds(rbase, NCHW * CH)],
                    vals_v.at[pl.ds(0, NCHW * CH)])

    @pl.when(extra == 1)
    def _():
        pltpu.sync_copy(idx_hbm.at[pl.ds(cbase + NCHW, 1)],
                        idx_v.at[pl.ds(NCHW, 1)])
        pltpu.sync_copy(vals_hbm.at[pl.ds(rbase + NCHW * CH, CH)],
                        vals_v.at[pl.ds(NCHW * CH, CH)])

    plsc.subcore_barrier()

    def fire(j, carry):
        pltpu.async_copy(vals_v.at[pl.ds(j * CH, CH)],
                         acc_sh.at[idx_v.at[j]], sem, add=True)
        if with_cnt:
            pltpu.async_copy(ones_v, cacc_sh.at[idx_v.at[j]], sem, add=True)
        return carry

    lax.fori_loop(0, nch, fire, 0)
    n_waits = nch * 2 if with_cnt else nch

    def drain(j, carry):
        pltpu.make_async_copy(vals_v.at[pl.ds(0, CH)],
                              acc_sh.at[idx_v.at[0]], sem).wait()
        return carry

    lax.fori_loop(0, n_waits, drain, 0)
    plsc.subcore_barrier()
    pltpu.sync_copy(acc_sh.at[pl.ds(s * RPS, RPS)],
                    out_hbm.at[pl.ds(c * NP + s * RPS, RPS)])
    if with_cnt:
        pltpu.sync_copy(cacc_sh.at[pl.ds(s * RPS, RPS)],
                        cout_hbm.at[pl.ds(c * NP + s * RPS, RPS)])


@functools.partial(
    pl.kernel,
    out_type=(jax.ShapeDtypeStruct((NC * NP, 16), jnp.float32),
              jax.ShapeDtypeStruct((NC * NP, 16), jnp.float32)),
    mesh=_MESH,
    scratch_types=[
        pltpu.VMEM((NCHW + 1, CH), jnp.int32),
        pltpu.VMEM(((NCHW + 1) * CH, 16), jnp.float32),
        pltpu.VMEM((CH, 16), jnp.float32),
        pltpu.VMEM_SHARED((NP, 16), jnp.float32),
        pltpu.VMEM_SHARED((NP, 16), jnp.float32),
        pltpu.SemaphoreType.DMA,
    ],
    compiler_params=_SC_PARAMS,
)
def _sc_scatter_cnt(idx_hbm, vals_hbm, zeros_hbm, ones_hbm, out_hbm,
                    cout_hbm, idx_v, vals_v, ones_v, acc_sh, cacc_sh, sem):
    _scatter_body(idx_hbm, vals_hbm, zeros_hbm, ones_hbm, out_hbm, cout_hbm,
                  idx_v, vals_v, ones_v, acc_sh, cacc_sh, sem, with_cnt=True)


@functools.partial(
    pl.kernel,
    out_type=jax.ShapeDtypeStruct((NC * NP, 16), jnp.float32),
    mesh=_MESH,
    scratch_types=[
        pltpu.VMEM((NCHW + 1, CH), jnp.int32),
        pltpu.VMEM(((NCHW + 1) * CH, 16), jnp.float32),
        pltpu.VMEM_SHARED((NP, 16), jnp.float32),
        pltpu.SemaphoreType.DMA,
    ],
    compiler_params=_SC_PARAMS,
)
def _sc_scatter(idx_hbm, vals_hbm, zeros_hbm, out_hbm, idx_v, vals_v, acc_sh,
                sem):
    _scatter_body(idx_hbm, vals_hbm, zeros_hbm, None, out_hbm, None,
                  idx_v, vals_v, None, acc_sh, None, sem, with_cnt=False)


def _msg_body(ea_ref, xs_ref, w1_ref, b1_ref, w2_ref, b2_ref, p_ref, q_ref,
              out_ref):
    ea = ea_ref[...]
    xs = xs_ref[...]
    h = jnp.maximum(
        jnp.dot(ea, w1_ref[...], preferred_element_type=jnp.float32)
        + b1_ref[...], 0.0)
    wf = jnp.dot(h, w2_ref[...], preferred_element_type=jnp.float32) \
        + b2_ref[...]
    xr = jnp.dot(xs, p_ref[...], preferred_element_type=jnp.float32)
    out_ref[...] = jnp.dot(xr * wf, q_ref[...],
                           preferred_element_type=jnp.float32)


def _edge_messages(ea, xs, w1, b1, w2, b2, ic, oc):
    grid = E // BE
    ii = jnp.arange(ic * oc) // oc
    oo = jnp.arange(ic * oc) % oc
    p_mat = (ii[None, :] == jnp.arange(ic)[:, None]).astype(jnp.float32)
    q_mat = (oo[:, None] == jnp.arange(oc)[None, :]).astype(jnp.float32)
    b1r = b1.reshape(1, -1)
    b2r = b2.reshape(1, -1)

    def fixed(a):
        return pl.BlockSpec(a.shape, lambda i: (0,) * a.ndim)

    return pl.pallas_call(
        _msg_body,
        grid=(grid,),
        in_specs=[
            pl.BlockSpec((BE, ic), lambda i: (i, 0)),
            pl.BlockSpec((BE, ic), lambda i: (i, 0)),
            fixed(w1), fixed(b1r), fixed(w2), fixed(b2r),
            fixed(p_mat), fixed(q_mat),
        ],
        out_specs=pl.BlockSpec((BE, oc), lambda i: (i, 0)),
        out_shape=jax.ShapeDtypeStruct((E, oc), jnp.float32),
    )(ea, xs, w1, b1r, w2, b2r, p_mat, q_mat)


def _node_body(x_ref, sum_ref, cnt_ref, root_ref, bias_ref, out_ref, *,
               do_relu):
    s = sum_ref[0:N] + sum_ref[NP:NP + N]
    cnt = cnt_ref[0:N] + cnt_ref[NP:NP + N]
    inv = 1.0 / jnp.maximum(cnt, 1.0)
    r = jnp.dot(x_ref[...], root_ref[...],
                preferred_element_type=jnp.float32)
    o = r + s * inv + bias_ref[...]
    if do_relu:
        o = jnp.maximum(o, 0.0)
    out_ref[...] = o


def _node_combine(x, summed, cnt, root, bias, do_relu):
    return pl.pallas_call(
        functools.partial(_node_body, do_relu=do_relu),
        out_shape=jax.ShapeDtypeStruct((N, 16), jnp.float32),
    )(x, summed, cnt, root, bias.reshape(1, -1))


def kernel(x, edge_index, edge_attr, nn1_w1, nn1_b1, nn1_w2, nn1_b2, root1,
           bias1, nn2_w1, nn2_b1, nn2_w2, nn2_b2, root2, bias2):
    src2 = edge_index[0].astype(jnp.int32).reshape(NCHT, CH)
    dst2 = edge_index[1].astype(jnp.int32).reshape(NCHT, CH)
    zeros_np = jnp.zeros((NP, 16), jnp.float32)
    ones_ch = jnp.ones((CH, 16), jnp.float32)

    xs = _sc_gather(x, src2)
    msg1 = _edge_messages(edge_attr, xs, nn1_w1, nn1_b1, nn1_w2, nn1_b2,
                          IN, HID)
    s1, c1 = _sc_scatter_cnt(dst2, msg1, zeros_np, ones_ch)
    h = _node_combine(x, s1, c1, root1, bias1, True)

    hs = _sc_gather(h, src2)
    msg2 = _edge_messages(edge_attr, hs, nn2_w1, nn2_b1, nn2_w2, nn2_b2,
                          HID, OUT)
    s2 = _sc_scatter(dst2, msg2, zeros_np)
    out = _node_combine(h, s2, c1, root2, bias2, False)
    return out
